# baseline (speedup 1.0000x reference)
import functools

import jax
import jax.numpy as jnp
from jax import lax
from jax.experimental import pallas as pl
from jax.experimental.pallas import tpu as pltpu
from jax.experimental.pallas import tpu_sc as plsc

NC = 2
NS = 16
NW = NC * NS
L = 16
CHUNK = 128


def _build_table(n_rels, h_ref, rel_ref, out_ref):
    hv = h_ref[...]
    for r in range(n_rels):
        out_ref[r] = jnp.maximum(hv + rel_ref[r], 0.0)


def _build_gidx(n_nodes, src_ref, eid_ref, out_ref):
    out_ref[...] = eid_ref[...] * n_nodes + src_ref[...]


def _finish(ps_ref, pc_ref, w_ref, b_ref, o_ref):
    s = ps_ref[0] + ps_ref[1]
    c = pc_ref[0] + pc_ref[1]
    red = s / jnp.maximum(c, 1.0)
    o_ref[...] = (
        lax.dot_general(red, w_ref[...], (((1,), (1,)), ((), ())),
                        preferred_element_type=jnp.float32)
        + b_ref[...]
    )


def _make_sc_scatter(n_nodes, dim, n_rels, cpw, acc_rows):
    assert cpw % 2 == 0
    rps = acc_rows // NS
    crows = acc_rows // L
    crps = crows // NS
    cxfers = crows // CHUNK
    assert rps % 8 == 0 and crows % CHUNK == 0 and crps % 8 == 0
    mesh = plsc.VectorSubcoreMesh(core_axis_name="c", subcore_axis_name="s")

    @functools.partial(
        pl.kernel,
        mesh=mesh,
        compiler_params=pltpu.CompilerParams(use_tc_tiling_on_sc=False,
                                             needs_layout_passes=False),
        out_type=[
            jax.ShapeDtypeStruct((NC, acc_rows, dim), jnp.float32),
            jax.ShapeDtypeStruct((NC, crows, L), jnp.float32),
        ],
        scratch_types=(
            [pltpu.VMEM((CHUNK,), jnp.int32) for _ in range(2)]
            + [pltpu.VMEM((CHUNK,), jnp.int32) for _ in range(2)]
            + [pltpu.VMEM((CHUNK, dim), jnp.float32) for _ in range(2)]
            + [
                pltpu.VMEM((crows, L), jnp.float32),
                pltpu.VMEM((cxfers, CHUNK), jnp.int32),
                pltpu.VMEM_SHARED((acc_rows, dim), jnp.float32),
                pltpu.VMEM_SHARED((crows, L), jnp.float32),
            ]
            + [pltpu.SemaphoreType.DMA for _ in range(5)]
        ),
    )
    def sc_kernel(r_hbm, gidx_hbm, dst_hbm, psum_hbm, pcnt_hbm,
                  i0, i1, d0, d1, rows0, rows1,
                  cnt_v, ident_v, acc_s, cnt_s,
                  sg0, sg1, si0, si1, ss):
        cid = lax.axis_index("c")
        sid = lax.axis_index("s")
        wid = sid * NC + cid

        zeros16 = jnp.zeros((L,), jnp.float32)
        ones16 = jnp.ones((L,), jnp.float32)
        iota16 = lax.iota(jnp.int32, L)

        def zr_body(k, _):
            rows0[k // (dim // L), pl.ds((k % (dim // L)) * L, L)] = zeros16
            return 0
        lax.fori_loop(0, CHUNK * (dim // L), zr_body, 0)

        def zc_body(k, _):
            cnt_v[k, :] = zeros16
            return 0
        lax.fori_loop(0, crows, zc_body, 0)

        for t in range(cxfers):
            for j in range(CHUNK // L):
                ident_v[t, pl.ds(j * L, L)] = iota16 + (t * CHUNK + j * L)

        base = sid * rps
        nfull, rem = rps // CHUNK, rps % CHUNK
        for k in range(nfull):
            pltpu.sync_copy(rows0, acc_s.at[pl.ds(base + k * CHUNK, CHUNK)])
        if rem:
            pltpu.sync_copy(rows0.at[pl.ds(0, rem)],
                            acc_s.at[pl.ds(base + nfull * CHUNK, rem)])
        pltpu.sync_copy(cnt_v.at[pl.ds(0, crps)],
                        cnt_s.at[pl.ds(sid * crps, crps)])
        plsc.subcore_barrier()

        pltpu.sync_copy(gidx_hbm.at[wid, 0], i0)
        pltpu.sync_copy(dst_hbm.at[wid, 0], d0)
        pltpu.sync_copy(gidx_hbm.at[wid, 1], i1)
        pltpu.sync_copy(dst_hbm.at[wid, 1], d1)

        def count_chunk(dref):
            for j in range(CHUNK // L):
                d = dref[pl.ds(j * L, L)]
                plsc.addupdate_scatter(
                    cnt_v, [lax.shift_right_logical(d, 4),
                            lax.bitwise_and(d, L - 1)], ones16)

        def body(c2, _):
            c = c2 * 2
            ga = pltpu.async_copy(r_hbm.at[i0], rows0, sg0)
            gb = pltpu.async_copy(r_hbm.at[i1], rows1, sg1)
            count_chunk(d0)
            count_chunk(d1)
            ga.wait()
            s1 = pltpu.async_copy(rows0, acc_s.at[d0], ss, add=True)
            s1.wait()
            ia = pltpu.async_copy(gidx_hbm.at[wid, c + 2], i0, si0)
            ib = pltpu.async_copy(dst_hbm.at[wid, c + 2], d0, si0)
            gb.wait()
            t1 = pltpu.async_copy(rows1, acc_s.at[d1], ss, add=True)
            t1.wait()
            ic = pltpu.async_copy(gidx_hbm.at[wid, c + 3], i1, si1)
            idd = pltpu.async_copy(dst_hbm.at[wid, c + 3], d1, si1)
            ia.wait()
            ib.wait()
            ic.wait()
            idd.wait()
            return 0
        lax.fori_loop(0, cpw // 2, body, 0)

        merges = [
            pltpu.async_copy(cnt_v.at[pl.ds(t * CHUNK, CHUNK)],
                             cnt_s.at[ident_v.at[t]], ss, add=True)
            for t in range(cxfers)
        ]
        for m in merges:
            m.wait()

        plsc.subcore_barrier()
        pltpu.sync_copy(acc_s.at[pl.ds(base, rps)],
                        psum_hbm.at[cid, pl.ds(base, rps)])
        pltpu.sync_copy(cnt_s.at[pl.ds(sid * crps, crps)],
                        pcnt_hbm.at[cid, pl.ds(sid * crps, crps)])

    return sc_kernel


def kernel(h, edge_index, edge_id, W, b, relvectors):
    n_nodes, dim = h.shape
    n_rels = relvectors.shape[0]
    n_edges = edge_index.shape[1]

    src = edge_index[0].astype(jnp.int32)
    dst = edge_index[1].astype(jnp.int32)
    eid = edge_id.astype(jnp.int32)

    cpw = -(-n_edges // (NW * CHUNK * 4)) * 4
    epad = NW * cpw * CHUNK
    pad = epad - n_edges
    acc_rows = -(-(n_nodes + (1 if pad else 0)) // (L * NS * 8)) * (L * NS * 8)

    if pad:
        par = jnp.arange(pad, dtype=jnp.int32)
        src = jnp.concatenate([src, par % n_nodes])
        eid = jnp.concatenate([eid, jnp.zeros((pad,), jnp.int32)])
        dst = jnp.concatenate([dst, n_nodes + par % (acc_rows - n_nodes)])

    nbs = 1000
    table = pl.pallas_call(
        functools.partial(_build_table, n_rels),
        grid=(n_nodes // nbs,),
        in_specs=[
            pl.BlockSpec((nbs, dim), lambda i: (i, 0)),
            pl.BlockSpec((n_rels, dim), lambda i: (0, 0)),
        ],
        out_specs=pl.BlockSpec((n_rels, nbs, dim), lambda i: (0, i, 0)),
        out_shape=jax.ShapeDtypeStruct((n_rels, n_nodes, dim), jnp.float32),
    )(h, relvectors).reshape(n_rels * n_nodes, dim)

    src2 = src.reshape(cpw, NW * CHUNK)
    eid2 = eid.reshape(cpw, NW * CHUNK)
    gidx = pl.pallas_call(
        functools.partial(_build_gidx, n_nodes),
        out_shape=jax.ShapeDtypeStruct((cpw, NW * CHUNK), jnp.int32),
    )(src2, eid2)

    gidx3 = jnp.concatenate(
        [gidx.reshape(NW, cpw, CHUNK),
         jnp.zeros((NW, 4, CHUNK), jnp.int32)], axis=1)
    dst3 = jnp.concatenate(
        [dst.reshape(NW, cpw, CHUNK),
         jnp.full((NW, 4, CHUNK), n_nodes, jnp.int32)], axis=1)

    psum, pcnt = _make_sc_scatter(n_nodes, dim, n_rels, cpw, acc_rows)(
        table, gidx3, dst3)

    grid_c = 8
    rbs = acc_rows // grid_c
    out = pl.pallas_call(
        _finish,
        grid=(grid_c,),
        in_specs=[
            pl.BlockSpec((NC, rbs, dim), lambda i: (0, i, 0)),
            pl.BlockSpec((NC, rbs, 1), lambda i: (0, i, 0)),
            pl.BlockSpec((dim, dim), lambda i: (0, 0)),
            pl.BlockSpec((1, dim), lambda i: (0, 0)),
        ],
        out_specs=pl.BlockSpec((rbs, dim), lambda i: (i, 0)),
        out_shape=jax.ShapeDtypeStruct((acc_rows, dim), jnp.float32),
    )(psum, pcnt.reshape(NC, acc_rows, 1), W, b.reshape(1, dim))
    return out[:n_nodes]


# speedup vs baseline: 1.0343x; 1.0343x over previous
import functools

import jax
import jax.numpy as jnp
from jax import lax
from jax.experimental import pallas as pl
from jax.experimental.pallas import tpu as pltpu
from jax.experimental.pallas import tpu_sc as plsc

NC = 2
NS = 16
NW = NC * NS
L = 16
CHUNK = 128
SL = 8


def _build_table(n_rels, h_ref, rel_ref, out_ref):
    hv = h_ref[...]
    for r in range(n_rels):
        out_ref[r] = jnp.maximum(hv + rel_ref[r], 0.0)


def _build_gidx(n_nodes, src_ref, eid_ref, out_ref):
    out_ref[...] = eid_ref[...] * n_nodes + src_ref[...]


def _finish(ps_ref, pc_ref, w_ref, b_ref, o_ref):
    s = ps_ref[0] + ps_ref[1]
    c = pc_ref[0] + pc_ref[1]
    red = s / jnp.maximum(c, 1.0)
    o_ref[...] = (
        lax.dot_general(red, w_ref[...], (((1,), (1,)), ((), ())),
                        preferred_element_type=jnp.float32)
        + b_ref[...]
    )


def _make_sc_scatter(n_nodes, dim, n_rels, cpw, acc_rows):
    assert cpw % SL == 0
    rps = acc_rows // NS
    crows = acc_rows // L
    crps = crows // NS
    cxfers = crows // CHUNK
    assert rps % 8 == 0 and crows % CHUNK == 0 and crps % 8 == 0
    mesh = plsc.VectorSubcoreMesh(core_axis_name="c", subcore_axis_name="s")

    @functools.partial(
        pl.kernel,
        mesh=mesh,
        compiler_params=pltpu.CompilerParams(use_tc_tiling_on_sc=False,
                                             needs_layout_passes=False),
        out_type=[
            jax.ShapeDtypeStruct((NC, acc_rows, dim), jnp.float32),
            jax.ShapeDtypeStruct((NC, crows, L), jnp.float32),
        ],
        scratch_types=(
            [pltpu.VMEM((SL, CHUNK), jnp.int32) for _ in range(2)]
            + [pltpu.VMEM((CHUNK, dim), jnp.float32) for _ in range(2)]
            + [
                pltpu.VMEM((crows, L), jnp.float32),
                pltpu.VMEM((cxfers, CHUNK), jnp.int32),
                pltpu.VMEM_SHARED((acc_rows, dim), jnp.float32),
                pltpu.VMEM_SHARED((crows, L), jnp.float32),
            ]
            + [pltpu.SemaphoreType.DMA for _ in range(3)]
        ),
    )
    def sc_kernel(r_hbm, gidx_hbm, dst_hbm, psum_hbm, pcnt_hbm,
                  gslab, dslab, rows0, rows1,
                  cnt_v, ident_v, acc_s, cnt_s,
                  sg0, sg1, ss):
        cid = lax.axis_index("c")
        sid = lax.axis_index("s")
        wid = sid * NC + cid

        zeros16 = jnp.zeros((L,), jnp.float32)
        ones16 = jnp.ones((L,), jnp.float32)
        iota16 = lax.iota(jnp.int32, L)

        def zr_body(k, _):
            rows0[k // (dim // L), pl.ds((k % (dim // L)) * L, L)] = zeros16
            return 0
        lax.fori_loop(0, CHUNK * (dim // L), zr_body, 0)

        def zc_body(k, _):
            cnt_v[k, :] = zeros16
            return 0
        lax.fori_loop(0, crows, zc_body, 0)

        for t in range(cxfers):
            for j in range(CHUNK // L):
                ident_v[t, pl.ds(j * L, L)] = iota16 + (t * CHUNK + j * L)

        base = sid * rps
        nfull, rem = rps // CHUNK, rps % CHUNK
        for k in range(nfull):
            pltpu.sync_copy(rows0, acc_s.at[pl.ds(base + k * CHUNK, CHUNK)])
        if rem:
            pltpu.sync_copy(rows0.at[pl.ds(0, rem)],
                            acc_s.at[pl.ds(base + nfull * CHUNK, rem)])
        pltpu.sync_copy(cnt_v.at[pl.ds(0, crps)],
                        cnt_s.at[pl.ds(sid * crps, crps)])
        plsc.subcore_barrier()

        def count_chunk(k):
            for j in range(CHUNK // L):
                d = dslab[k, pl.ds(j * L, L)]
                plsc.addupdate_scatter(
                    cnt_v, [lax.shift_right_logical(d, 4),
                            lax.bitwise_and(d, L - 1)], ones16)

        def body(it, _):
            pltpu.sync_copy(gidx_hbm.at[wid, it], gslab)
            pltpu.sync_copy(dst_hbm.at[wid, it], dslab)
            for k in range(0, SL, 2):
                ga = pltpu.async_copy(r_hbm.at[gslab.at[k]], rows0, sg0)
                gb = pltpu.async_copy(r_hbm.at[gslab.at[k + 1]], rows1, sg1)
                count_chunk(k)
                count_chunk(k + 1)
                ga.wait()
                s1 = pltpu.async_copy(rows0, acc_s.at[dslab.at[k]], ss,
                                      add=True)
                s1.wait()
                gb.wait()
                t1 = pltpu.async_copy(rows1, acc_s.at[dslab.at[k + 1]], ss,
                                      add=True)
                t1.wait()
            return 0
        lax.fori_loop(0, cpw // SL, body, 0)

        merges = [
            pltpu.async_copy(cnt_v.at[pl.ds(t * CHUNK, CHUNK)],
                             cnt_s.at[ident_v.at[t]], ss, add=True)
            for t in range(cxfers)
        ]
        for m in merges:
            m.wait()

        plsc.subcore_barrier()
        pltpu.sync_copy(acc_s.at[pl.ds(base, rps)],
                        psum_hbm.at[cid, pl.ds(base, rps)])
        pltpu.sync_copy(cnt_s.at[pl.ds(sid * crps, crps)],
                        pcnt_hbm.at[cid, pl.ds(sid * crps, crps)])

    return sc_kernel


def kernel(h, edge_index, edge_id, W, b, relvectors):
    n_nodes, dim = h.shape
    n_rels = relvectors.shape[0]
    n_edges = edge_index.shape[1]

    src = edge_index[0].astype(jnp.int32)
    dst = edge_index[1].astype(jnp.int32)
    eid = edge_id.astype(jnp.int32)

    cpw = -(-n_edges // (NW * CHUNK * SL)) * SL
    epad = NW * cpw * CHUNK
    pad = epad - n_edges
    acc_rows = -(-(n_nodes + (1 if pad else 0)) // (L * NS * 8)) * (L * NS * 8)

    if pad:
        par = jnp.arange(pad, dtype=jnp.int32)
        src = jnp.concatenate([src, par % n_nodes])
        eid = jnp.concatenate([eid, jnp.zeros((pad,), jnp.int32)])
        dst = jnp.concatenate([dst, n_nodes + par % (acc_rows - n_nodes)])

    nbs = 1000
    table = pl.pallas_call(
        functools.partial(_build_table, n_rels),
        grid=(n_nodes // nbs,),
        in_specs=[
            pl.BlockSpec((nbs, dim), lambda i: (i, 0)),
            pl.BlockSpec((n_rels, dim), lambda i: (0, 0)),
        ],
        out_specs=pl.BlockSpec((n_rels, nbs, dim), lambda i: (0, i, 0)),
        out_shape=jax.ShapeDtypeStruct((n_rels, n_nodes, dim), jnp.float32),
    )(h, relvectors).reshape(n_rels * n_nodes, dim)

    src2 = src.reshape(cpw, NW * CHUNK)
    eid2 = eid.reshape(cpw, NW * CHUNK)
    gidx = pl.pallas_call(
        functools.partial(_build_gidx, n_nodes),
        out_shape=jax.ShapeDtypeStruct((cpw, NW * CHUNK), jnp.int32),
    )(src2, eid2)

    gidx3 = gidx.reshape(NW, cpw // SL, SL, CHUNK)
    dst3 = dst.reshape(NW, cpw // SL, SL, CHUNK)

    psum, pcnt = _make_sc_scatter(n_nodes, dim, n_rels, cpw, acc_rows)(
        table, gidx3, dst3)

    grid_c = 8
    rbs = acc_rows // grid_c
    out = pl.pallas_call(
        _finish,
        grid=(grid_c,),
        in_specs=[
            pl.BlockSpec((NC, rbs, dim), lambda i: (0, i, 0)),
            pl.BlockSpec((NC, rbs, 1), lambda i: (0, i, 0)),
            pl.BlockSpec((dim, dim), lambda i: (0, 0)),
            pl.BlockSpec((1, dim), lambda i: (0, 0)),
        ],
        out_specs=pl.BlockSpec((rbs, dim), lambda i: (i, 0)),
        out_shape=jax.ShapeDtypeStruct((acc_rows, dim), jnp.float32),
    )(psum, pcnt.reshape(NC, acc_rows, 1), W, b.reshape(1, dim))
    return out[:n_nodes]


# speedup vs baseline: 1.1863x; 1.1470x over previous
import functools

import jax
import jax.numpy as jnp
from jax import lax
from jax.experimental import pallas as pl
from jax.experimental.pallas import tpu as pltpu
from jax.experimental.pallas import tpu_sc as plsc

NC = 2
NS = 16
NW = NC * NS
L = 16
CHUNK = 128
SL = 8


def _build_table(n_rels, h_ref, rel_ref, out_ref):
    hv = h_ref[...]
    for r in range(n_rels):
        out_ref[r] = jnp.maximum(hv + rel_ref[r], 0.0)


def _build_gidx(n_nodes, src_ref, eid_ref, out_ref):
    out_ref[...] = eid_ref[...] * n_nodes + src_ref[...]


def _finish(ps_ref, pc_ref, w_ref, b_ref, o_ref):
    s = ps_ref[0] + ps_ref[1]
    c = pc_ref[0] + pc_ref[1]
    red = s / jnp.maximum(c, 1.0)
    o_ref[...] = (
        lax.dot_general(red, w_ref[...], (((1,), (1,)), ((), ())),
                        preferred_element_type=jnp.float32)
        + b_ref[...]
    )


def _make_sc_scatter(n_nodes, dim, n_rels, cpw, acc_rows):
    assert cpw % SL == 0
    rps = acc_rows // NS
    crows = acc_rows // L
    crps = crows // NS
    cxfers = crows // CHUNK
    assert rps % 8 == 0 and crows % CHUNK == 0 and crps % 8 == 0
    mesh = plsc.VectorSubcoreMesh(core_axis_name="c", subcore_axis_name="s")

    @functools.partial(
        pl.kernel,
        mesh=mesh,
        compiler_params=pltpu.CompilerParams(use_tc_tiling_on_sc=False,
                                             needs_layout_passes=False),
        out_type=[
            jax.ShapeDtypeStruct((NC, acc_rows, dim), jnp.float32),
            jax.ShapeDtypeStruct((NC, crows, L), jnp.float32),
        ],
        scratch_types=(
            [pltpu.VMEM((SL, CHUNK), jnp.int32) for _ in range(2)]
            + [pltpu.VMEM((CHUNK, dim), jnp.float32) for _ in range(2)]
            + [
                pltpu.VMEM((crows, L), jnp.float32),
                pltpu.VMEM((cxfers, CHUNK), jnp.int32),
                pltpu.VMEM_SHARED((acc_rows, dim), jnp.float32),
                pltpu.VMEM_SHARED((crows, L), jnp.float32),
            ]
            + [pltpu.SemaphoreType.DMA for _ in range(3)]
        ),
    )
    def sc_kernel(r_hbm, gidx_hbm, dst_hbm, psum_hbm, pcnt_hbm,
                  gslab, dslab, rows0, rows1,
                  cnt_v, ident_v, acc_s, cnt_s,
                  sg0, sg1, ss):
        cid = lax.axis_index("c")
        sid = lax.axis_index("s")
        wid = sid * NC + cid

        zeros16 = jnp.zeros((L,), jnp.float32)
        ones16 = jnp.ones((L,), jnp.float32)
        iota16 = lax.iota(jnp.int32, L)

        def zr_body(k, _):
            rows0[k // (dim // L), pl.ds((k % (dim // L)) * L, L)] = zeros16
            return 0
        lax.fori_loop(0, CHUNK * (dim // L), zr_body, 0)

        def zc_body(k, _):
            cnt_v[k, :] = zeros16
            return 0
        lax.fori_loop(0, crows, zc_body, 0)

        for t in range(cxfers):
            for j in range(CHUNK // L):
                ident_v[t, pl.ds(j * L, L)] = iota16 + (t * CHUNK + j * L)

        base = sid * rps
        nfull, rem = rps // CHUNK, rps % CHUNK
        for k in range(nfull):
            pltpu.sync_copy(rows0, acc_s.at[pl.ds(base + k * CHUNK, CHUNK)])
        if rem:
            pltpu.sync_copy(rows0.at[pl.ds(0, rem)],
                            acc_s.at[pl.ds(base + nfull * CHUNK, rem)])
        pltpu.sync_copy(cnt_v.at[pl.ds(0, crps)],
                        cnt_s.at[pl.ds(sid * crps, crps)])
        plsc.subcore_barrier()

        def count_chunk(k):
            for j in range(CHUNK // L):
                d = dslab[k, pl.ds(j * L, L)]
                plsc.addupdate_scatter(
                    cnt_v, [lax.shift_right_logical(d, 4),
                            lax.bitwise_and(d, L - 1)], ones16)

        rows = [rows0, rows1]
        sgs = [sg0, sg1]

        def body(it, _):
            pltpu.sync_copy(gidx_hbm.at[wid, it], gslab)
            pltpu.sync_copy(dst_hbm.at[wid, it], dslab)
            g = [None] * SL
            g[0] = pltpu.async_copy(r_hbm.at[gslab.at[0]], rows[0], sgs[0])
            g[1] = pltpu.async_copy(r_hbm.at[gslab.at[1]], rows[1], sgs[1])
            for k in range(SL):
                count_chunk(k)
            for k in range(SL):
                g[k].wait()
                s = pltpu.async_copy(rows[k % 2], acc_s.at[dslab.at[k]], ss,
                                     add=True)
                s.wait()
                if k + 2 < SL:
                    g[k + 2] = pltpu.async_copy(r_hbm.at[gslab.at[k + 2]],
                                                rows[k % 2], sgs[k % 2])
            return 0
        lax.fori_loop(0, cpw // SL, body, 0)

        merges = [
            pltpu.async_copy(cnt_v.at[pl.ds(t * CHUNK, CHUNK)],
                             cnt_s.at[ident_v.at[t]], ss, add=True)
            for t in range(cxfers)
        ]
        for m in merges:
            m.wait()

        plsc.subcore_barrier()
        pltpu.sync_copy(acc_s.at[pl.ds(base, rps)],
                        psum_hbm.at[cid, pl.ds(base, rps)])
        pltpu.sync_copy(cnt_s.at[pl.ds(sid * crps, crps)],
                        pcnt_hbm.at[cid, pl.ds(sid * crps, crps)])

    return sc_kernel


def kernel(h, edge_index, edge_id, W, b, relvectors):
    n_nodes, dim = h.shape
    n_rels = relvectors.shape[0]
    n_edges = edge_index.shape[1]

    src = edge_index[0].astype(jnp.int32)
    dst = edge_index[1].astype(jnp.int32)
    eid = edge_id.astype(jnp.int32)

    cpw = -(-n_edges // (NW * CHUNK * SL)) * SL
    epad = NW * cpw * CHUNK
    pad = epad - n_edges
    acc_rows = -(-(n_nodes + (1 if pad else 0)) // (L * NS * 8)) * (L * NS * 8)

    if pad:
        par = jnp.arange(pad, dtype=jnp.int32)
        src = jnp.concatenate([src, par % n_nodes])
        eid = jnp.concatenate([eid, jnp.zeros((pad,), jnp.int32)])
        dst = jnp.concatenate([dst, n_nodes + par % (acc_rows - n_nodes)])

    nbs = 1000
    table = pl.pallas_call(
        functools.partial(_build_table, n_rels),
        grid=(n_nodes // nbs,),
        in_specs=[
            pl.BlockSpec((nbs, dim), lambda i: (i, 0)),
            pl.BlockSpec((n_rels, dim), lambda i: (0, 0)),
        ],
        out_specs=pl.BlockSpec((n_rels, nbs, dim), lambda i: (0, i, 0)),
        out_shape=jax.ShapeDtypeStruct((n_rels, n_nodes, dim), jnp.float32),
    )(h, relvectors).reshape(n_rels * n_nodes, dim)

    src2 = src.reshape(cpw, NW * CHUNK)
    eid2 = eid.reshape(cpw, NW * CHUNK)
    gidx = pl.pallas_call(
        functools.partial(_build_gidx, n_nodes),
        out_shape=jax.ShapeDtypeStruct((cpw, NW * CHUNK), jnp.int32),
    )(src2, eid2)

    gidx3 = gidx.reshape(NW, cpw // SL, SL, CHUNK)
    dst3 = dst.reshape(NW, cpw // SL, SL, CHUNK)

    psum, pcnt = _make_sc_scatter(n_nodes, dim, n_rels, cpw, acc_rows)(
        table, gidx3, dst3)

    grid_c = 8
    rbs = acc_rows // grid_c
    out = pl.pallas_call(
        _finish,
        grid=(grid_c,),
        in_specs=[
            pl.BlockSpec((NC, rbs, dim), lambda i: (0, i, 0)),
            pl.BlockSpec((NC, rbs, 1), lambda i: (0, i, 0)),
            pl.BlockSpec((dim, dim), lambda i: (0, 0)),
            pl.BlockSpec((1, dim), lambda i: (0, 0)),
        ],
        out_specs=pl.BlockSpec((rbs, dim), lambda i: (i, 0)),
        out_shape=jax.ShapeDtypeStruct((acc_rows, dim), jnp.float32),
    )(psum, pcnt.reshape(NC, acc_rows, 1), W, b.reshape(1, dim))
    return out[:n_nodes]


# speedup vs baseline: 1.2329x; 1.0393x over previous
import functools

import jax
import jax.numpy as jnp
from jax import lax
from jax.experimental import pallas as pl
from jax.experimental.pallas import tpu as pltpu
from jax.experimental.pallas import tpu_sc as plsc

NC = 2
NS = 16
NW = NC * NS
L = 16
CHUNK = 128
SL = 8


def _prep(n_rels, n_nodes, h_ref, rel_ref, src_ref, eid_ref,
          tab_ref, gidx_ref):
    hv = h_ref[...]
    for r in range(n_rels):
        tab_ref[r] = jnp.maximum(hv + rel_ref[r], 0.0)
    gidx_ref[...] = eid_ref[...] * n_nodes + src_ref[...]


def _finish(ps_ref, pc_ref, w_ref, b_ref, o_ref):
    s = ps_ref[0] + ps_ref[1]
    c = pc_ref[0] + pc_ref[1]
    red = s / jnp.maximum(c, 1.0)
    o_ref[...] = (
        lax.dot_general(red, w_ref[...], (((1,), (1,)), ((), ())),
                        preferred_element_type=jnp.float32)
        + b_ref[...]
    )


def _make_sc_scatter(n_nodes, dim, n_rels, cpw, acc_rows):
    assert cpw % SL == 0
    rps = acc_rows // NS
    crows = acc_rows // L
    crps = crows // NS
    cxfers = crows // CHUNK
    assert rps % 8 == 0 and crows % CHUNK == 0 and crps % 8 == 0
    mesh = plsc.VectorSubcoreMesh(core_axis_name="c", subcore_axis_name="s")

    @functools.partial(
        pl.kernel,
        mesh=mesh,
        compiler_params=pltpu.CompilerParams(use_tc_tiling_on_sc=False,
                                             needs_layout_passes=False),
        out_type=[
            jax.ShapeDtypeStruct((NC, acc_rows, dim), jnp.float32),
            jax.ShapeDtypeStruct((NC, crows, L), jnp.float32),
        ],
        scratch_types=(
            [pltpu.VMEM((SL, CHUNK), jnp.int32) for _ in range(2)]
            + [pltpu.VMEM((CHUNK, dim), jnp.float32) for _ in range(2)]
            + [
                pltpu.VMEM((crows, L), jnp.float32),
                pltpu.VMEM((cxfers, CHUNK), jnp.int32),
                pltpu.VMEM_SHARED((acc_rows, dim), jnp.float32),
                pltpu.VMEM_SHARED((crows, L), jnp.float32),
            ]
            + [pltpu.SemaphoreType.DMA for _ in range(3)]
        ),
    )
    def sc_kernel(r_hbm, gidx_hbm, dst_hbm, psum_hbm, pcnt_hbm,
                  gslab, dslab, rows0, rows1,
                  cnt_v, ident_v, acc_s, cnt_s,
                  sg0, sg1, ss):
        cid = lax.axis_index("c")
        sid = lax.axis_index("s")
        wid = sid * NC + cid

        zeros16 = jnp.zeros((L,), jnp.float32)
        ones16 = jnp.ones((L,), jnp.float32)
        iota16 = lax.iota(jnp.int32, L)

        def zr_body(k, _):
            for j in range(dim // L):
                rows0[k, pl.ds(j * L, L)] = zeros16
            return 0
        lax.fori_loop(0, CHUNK, zr_body, 0)

        def zc_body(k, _):
            for j in range(8):
                cnt_v[k * 8 + j, :] = zeros16
            return 0
        lax.fori_loop(0, crows // 8, zc_body, 0)

        for t in range(cxfers):
            for j in range(CHUNK // L):
                ident_v[t, pl.ds(j * L, L)] = iota16 + (t * CHUNK + j * L)

        base = sid * rps
        nfull, rem = rps // CHUNK, rps % CHUNK
        for k in range(nfull):
            pltpu.sync_copy(rows0, acc_s.at[pl.ds(base + k * CHUNK, CHUNK)])
        if rem:
            pltpu.sync_copy(rows0.at[pl.ds(0, rem)],
                            acc_s.at[pl.ds(base + nfull * CHUNK, rem)])
        pltpu.sync_copy(cnt_v.at[pl.ds(0, crps)],
                        cnt_s.at[pl.ds(sid * crps, crps)])
        plsc.subcore_barrier()

        def count_chunk(k):
            for j in range(CHUNK // L):
                d = dslab[k, pl.ds(j * L, L)]
                plsc.addupdate_scatter(
                    cnt_v, [lax.shift_right_logical(d, 4),
                            lax.bitwise_and(d, L - 1)], ones16)

        rows = [rows0, rows1]
        sgs = [sg0, sg1]

        def body(it, _):
            pltpu.sync_copy(gidx_hbm.at[wid, it], gslab)
            pltpu.sync_copy(dst_hbm.at[wid, it], dslab)
            g = [None] * SL
            g[0] = pltpu.async_copy(r_hbm.at[gslab.at[0]], rows[0], sgs[0])
            g[1] = pltpu.async_copy(r_hbm.at[gslab.at[1]], rows[1], sgs[1])
            for k in range(SL):
                count_chunk(k)
            for k in range(SL):
                g[k].wait()
                s = pltpu.async_copy(rows[k % 2], acc_s.at[dslab.at[k]], ss,
                                     add=True)
                s.wait()
                if k + 2 < SL:
                    g[k + 2] = pltpu.async_copy(r_hbm.at[gslab.at[k + 2]],
                                                rows[k % 2], sgs[k % 2])
            return 0
        lax.fori_loop(0, cpw // SL, body, 0)

        merges = [
            pltpu.async_copy(cnt_v.at[pl.ds(t * CHUNK, CHUNK)],
                             cnt_s.at[ident_v.at[t]], ss, add=True)
            for t in range(cxfers)
        ]
        for m in merges:
            m.wait()

        plsc.subcore_barrier()
        pltpu.sync_copy(acc_s.at[pl.ds(base, rps)],
                        psum_hbm.at[cid, pl.ds(base, rps)])
        pltpu.sync_copy(cnt_s.at[pl.ds(sid * crps, crps)],
                        pcnt_hbm.at[cid, pl.ds(sid * crps, crps)])

    return sc_kernel


def kernel(h, edge_index, edge_id, W, b, relvectors):
    n_nodes, dim = h.shape
    n_rels = relvectors.shape[0]
    n_edges = edge_index.shape[1]

    src = edge_index[0].astype(jnp.int32)
    dst = edge_index[1].astype(jnp.int32)
    eid = edge_id.astype(jnp.int32)

    cpw = -(-n_edges // (NW * CHUNK * SL)) * SL
    epad = NW * cpw * CHUNK
    pad = epad - n_edges
    acc_rows = -(-(n_nodes + (1 if pad else 0)) // (L * NS * 8)) * (L * NS * 8)

    if pad:
        par = jnp.arange(pad, dtype=jnp.int32)
        src = jnp.concatenate([src, par % n_nodes])
        eid = jnp.concatenate([eid, jnp.zeros((pad,), jnp.int32)])
        dst = jnp.concatenate([dst, n_nodes + par % (acc_rows - n_nodes)])

    grid_a = 10
    nbs = n_nodes // grid_a
    ebs = cpw // grid_a
    src2 = src.reshape(cpw, NW * CHUNK)
    eid2 = eid.reshape(cpw, NW * CHUNK)
    table, gidx = pl.pallas_call(
        functools.partial(_prep, n_rels, n_nodes),
        grid=(grid_a,),
        in_specs=[
            pl.BlockSpec((nbs, dim), lambda i: (i, 0)),
            pl.BlockSpec((n_rels, dim), lambda i: (0, 0)),
            pl.BlockSpec((ebs, NW * CHUNK), lambda i: (i, 0)),
            pl.BlockSpec((ebs, NW * CHUNK), lambda i: (i, 0)),
        ],
        out_specs=[
            pl.BlockSpec((n_rels, nbs, dim), lambda i: (0, i, 0)),
            pl.BlockSpec((ebs, NW * CHUNK), lambda i: (i, 0)),
        ],
        out_shape=[
            jax.ShapeDtypeStruct((n_rels, n_nodes, dim), jnp.float32),
            jax.ShapeDtypeStruct((cpw, NW * CHUNK), jnp.int32),
        ],
    )(h, relvectors, src2, eid2)
    table = table.reshape(n_rels * n_nodes, dim)

    gidx3 = gidx.reshape(NW, cpw // SL, SL, CHUNK)
    dst3 = dst.reshape(NW, cpw // SL, SL, CHUNK)

    psum, pcnt = _make_sc_scatter(n_nodes, dim, n_rels, cpw, acc_rows)(
        table, gidx3, dst3)

    grid_c = 8
    rbs = acc_rows // grid_c
    out = pl.pallas_call(
        _finish,
        grid=(grid_c,),
        in_specs=[
            pl.BlockSpec((NC, rbs, dim), lambda i: (0, i, 0)),
            pl.BlockSpec((NC, rbs, 1), lambda i: (0, i, 0)),
            pl.BlockSpec((dim, dim), lambda i: (0, 0)),
            pl.BlockSpec((1, dim), lambda i: (0, 0)),
        ],
        out_specs=pl.BlockSpec((rbs, dim), lambda i: (i, 0)),
        out_shape=jax.ShapeDtypeStruct((acc_rows, dim), jnp.float32),
    )(psum, pcnt.reshape(NC, acc_rows, 1), W, b.reshape(1, dim))
    return out[:n_nodes]
